# baseline (speedup 1.0000x reference)
import functools

import jax
import jax.numpy as jnp
import numpy as np
from jax.experimental import pallas as pl
from jax.experimental.pallas import tpu as pltpu

_EPS = 1e-5
_VMEM = 60 * 1024 * 1024


def _make_scatter_matrix():
    s = np.zeros((196 * 16, 784), np.float32)
    for ih in range(14):
        for iw in range(14):
            for kh in range(4):
                for kw in range(4):
                    oh = 2 * ih + kh - 1
                    ow = 2 * iw + kw - 1
                    if 0 <= oh < 28 and 0 <= ow < 28:
                        s[(ih * 14 + iw) * 16 + kh * 4 + kw, oh * 28 + ow] = 1.0
    return s


_S4 = _make_scatter_matrix()


def _bn_scale_shift(x_full, gamma, beta):
    n = x_full.shape[0]
    mean = jnp.sum(x_full, axis=0, keepdims=True) * (1.0 / n)
    ex2 = jnp.sum(x_full * x_full, axis=0, keepdims=True) * (1.0 / n)
    inv = jax.lax.rsqrt(ex2 - mean * mean + _EPS)
    scale = inv * gamma
    return scale, beta - mean * scale


def _phase_taps(ph, hin):
    if ph == 0:
        return [(1, 0, 0, hin), (3, 1, 0, hin - 1)]
    return [(2, 0, 0, hin), (0, 0, 1, hin - 1)]


def _stage1_kernel(x_ref, w1_ref, b1_ref, t1_ref, o_ref, hb_ref):
    j = pl.program_id(0)

    @pl.when(j == 0)
    def _():
        h = jnp.dot(x_ref[...].astype(jnp.bfloat16),
                    w1_ref[...].astype(jnp.bfloat16),
                    preferred_element_type=jnp.float32) + b1_ref[...]
        hb_ref[...] = h.astype(jnp.bfloat16)

    o_ref[...] = jnp.dot(hb_ref[...], t1_ref[...].astype(jnp.bfloat16),
                         preferred_element_type=jnp.float32)


def _stage2_kernel(y1_ref, g_ref, b_ref, w2_ref, o_ref, wb_ref, ss_ref,
                   acc_ref, *, bb):
    i = pl.program_id(0)

    @pl.when(i == 0)
    def _():
        wb_ref[...] = (w2_ref[...] * 1024.0).astype(jnp.float8_e4m3fn)
        scale, shift = _bn_scale_shift(y1_ref[...], g_ref[...], b_ref[...])
        ss_ref[0:1, :] = scale
        ss_ref[1:2, :] = shift

    sl = y1_ref[pl.ds(i * (bb * 16), bb * 16), :]
    a = jnp.tanh(sl * ss_ref[0:1, :] + ss_ref[1:2, :])
    y = jnp.dot((a * 64.0).astype(jnp.float8_e4m3fn), wb_ref[...],
                preferred_element_type=jnp.float32) * (1.0 / (64.0 * 1024.0))
    acc_ref[...] = jnp.zeros_like(acc_ref)
    for kh in range(4):
        for kw in range(4):
            tap = y[:, (kh * 4 + kw) * 256:(kh * 4 + kw + 1) * 256]
            acc_ref[:, kh:kh + 4, kw:kw + 4, :] += tap.reshape(bb, 4, 4, 256)
    o_ref[...] = acc_ref[...].reshape(bb * 49, 256)


_P_SH = {0: ((0, 1), (1, 3)), 1: ((0, 2), (-1, 0))}


def _shift_rows(x, s, axis):
    if s == 0:
        return x
    n = x.shape[axis]
    pad = [(0, 0)] * x.ndim
    if s > 0:
        pad[axis] = (s, 0)
        sl = [slice(None)] * x.ndim
        sl[axis] = slice(0, n - s)
    else:
        pad[axis] = (0, -s)
        sl = [slice(None)] * x.ndim
        sl[axis] = slice(-s, n)
    return jnp.pad(x[tuple(sl)], pad)


def _stage3_kernel(y2_ref, g_ref, b_ref, w3_ref, o_ref, wb_ref, ss_ref, *, bb):
    i = pl.program_id(0)

    @pl.when(i == 0)
    def _():
        w3b = (w3_ref[...] * 1024.0).astype(jnp.float8_e4m3fn)
        for p, (hp, wp) in enumerate(((0, 0), (0, 1), (1, 0), (1, 1))):
            blocks = []
            for _, kh in _P_SH[hp]:
                for _, kw in _P_SH[wp]:
                    t = kh * 4 + kw
                    blocks.append(w3b[:, t * 128:(t + 1) * 128])
            wb_ref[:, p * 128:(p + 1) * 128] = jnp.concatenate(blocks, axis=0)
        scale, shift = _bn_scale_shift(y2_ref[...], g_ref[...], b_ref[...])
        ss_ref[0:1, :] = scale
        ss_ref[1:2, :] = shift

    sl = y2_ref[pl.ds(i * (bb * 49), bb * 49), :]
    a = (jnp.tanh(sl * ss_ref[0:1, :] + ss_ref[1:2, :])
         * 64.0).astype(jnp.float8_e4m3fn)
    x4 = a.reshape(bb, 7, 7, 256)
    q = []
    for hp in range(2):
        p = []
        for wp in range(2):
            acat = jnp.concatenate(
                [_shift_rows(_shift_rows(x4, sh, 1), sw, 2)
                 for sh, _ in _P_SH[hp] for sw, _ in _P_SH[wp]],
                axis=-1).reshape(bb * 49, 1024)
            pidx = hp * 2 + wp
            p.append((jnp.dot(acat, wb_ref[:, pidx * 128:(pidx + 1) * 128],
                              preferred_element_type=jnp.float32)
                      * (1.0 / (64.0 * 1024.0))).reshape(bb, 7, 7, 128))
        q.append(jnp.stack(p, axis=3).reshape(bb, 7, 14, 128))
    out = jnp.stack(q, axis=2).reshape(bb, 14, 14, 128)
    o_ref[...] = out.reshape(bb * 196, 128)


def _stage4_kernel(y3_ref, g_ref, b_ref, w4_ref, o_ref, wb_ref, ss_ref, *, bb):
    i = pl.program_id(0)

    @pl.when(i == 0)
    def _():
        wb_ref[...] = w4_ref[...].astype(jnp.bfloat16)
        scale, shift = _bn_scale_shift(y3_ref[...], g_ref[...], b_ref[...])
        ss_ref[0:1, :] = scale
        ss_ref[1:2, :] = shift

    sl = y3_ref[pl.ds(i * (bb * 196), bb * 196), :]
    a = jnp.tanh(sl * ss_ref[0:1, :] + ss_ref[1:2, :])
    y = jnp.dot(a.astype(jnp.bfloat16), wb_ref[...],
                preferred_element_type=jnp.float32)[:, :16]
    o_ref[...] = y.astype(jnp.bfloat16)


def _mid_kernel(y1_ref, g1_ref, b1_ref, w2f8_ref, g2_ref, b2_ref, wb3_ref,
                g3_ref, b3_ref, wb4_ref, o_ref,
                ss1_ref, acc2_ref, y2s_ref,
                ss2_ref, y3s_ref, ss3_ref, *, B):
    j = pl.program_id(0)
    bbA = B // 8
    bbC = B // 4

    @pl.when(j < 8)
    def _stage2():
        @pl.when(j == 0)
        def _():
            scale, shift = _bn_scale_shift(y1_ref[...], g1_ref[...],
                                           b1_ref[...])
            ss1_ref[0:1, :] = scale
            ss1_ref[1:2, :] = shift

        sl = y1_ref[pl.ds(j * (bbA * 16), bbA * 16), :]
        a = jnp.tanh(sl * ss1_ref[0:1, :] + ss1_ref[1:2, :])
        y = jnp.dot((a * 64.0).astype(jnp.float8_e4m3fn), w2f8_ref[...],
                    preferred_element_type=jnp.float32) * (1.0 / 65536.0)
        acc2_ref[...] = jnp.zeros_like(acc2_ref)
        for kh in range(4):
            for kw in range(4):
                tap = y[:, (kh * 4 + kw) * 256:(kh * 4 + kw + 1) * 256]
                acc2_ref[:, kh:kh + 4, kw:kw + 4, :] += tap.reshape(
                    bbA, 4, 4, 256)
        y2s_ref[pl.ds(j * bbA, bbA)] = acc2_ref[...].reshape(
            bbA, 49, 256).astype(jnp.bfloat16)

    @pl.when((j >= 8) & (j < 16))
    def _stage3():
        @pl.when(j == 8)
        def _():
            scale, shift = _bn_scale_shift(
                y2s_ref[...].reshape(B * 49, 256).astype(jnp.float32),
                g2_ref[...], b2_ref[...])
            ss2_ref[0:1, :] = scale
            ss2_ref[1:2, :] = shift

        i = j - 8
        sl = y2s_ref[pl.ds(i * bbA, bbA)].reshape(
            bbA * 49, 256).astype(jnp.float32)
        a = (jnp.tanh(sl * ss2_ref[0:1, :] + ss2_ref[1:2, :])
             * 64.0).astype(jnp.float8_e4m3fn)
        x4 = a.reshape(bbA, 7, 7, 256)
        q = []
        for hp in range(2):
            p = []
            for wp in range(2):
                acat = jnp.concatenate(
                    [_shift_rows(_shift_rows(x4, sh, 1), sw, 2)
                     for sh, _ in _P_SH[hp] for sw, _ in _P_SH[wp]],
                    axis=-1).reshape(bbA * 49, 1024)
                pidx = hp * 2 + wp
                p.append((jnp.dot(acat,
                                  wb3_ref[:, pidx * 128:(pidx + 1) * 128],
                                  preferred_element_type=jnp.float32)
                          * (1.0 / 65536.0)).reshape(bbA, 7, 7, 128))
            q.append(jnp.stack(p, axis=3).reshape(bbA, 7, 14, 128))
        out = jnp.stack(q, axis=2).reshape(bbA, 14, 14, 128)
        y3s_ref[pl.ds(i * bbA, bbA)] = out.reshape(
            bbA, 196, 128).astype(jnp.bfloat16)

    @pl.when(j >= 16)
    def _stage4():
        @pl.when(j == 16)
        def _():
            scale, shift = _bn_scale_shift(
                y3s_ref[...].reshape(B * 196, 128).astype(jnp.float32),
                g3_ref[...], b3_ref[...])
            ss3_ref[0:1, :] = scale
            ss3_ref[1:2, :] = shift

        i = j - 16
        sl = y3s_ref[pl.ds(i * bbC, bbC)].reshape(
            bbC * 196, 128).astype(jnp.float32)
        a = jnp.tanh(sl * ss3_ref[0:1, :] + ss3_ref[1:2, :])
        y = jnp.dot(a.astype(jnp.bfloat16), wb4_ref[...],
                    preferred_element_type=jnp.float32)[:, :16]
        o_ref[...] = y.astype(jnp.bfloat16)


def _stage5_kernel(yl_ref, s4_ref, b4_ref, o_ref):
    z = jnp.dot(yl_ref[...], s4_ref[...],
                preferred_element_type=jnp.float32) + b4_ref[0, 0]
    ex = jnp.exp(-jnp.abs(z))
    o_ref[...] = jnp.where(z >= 0, 1.0 / (1.0 + ex), ex / (1.0 + ex))


def _pcall(kfn, grid, in_specs, args, out_shape, out_spec,
           out_dtype=jnp.float32, scratch=()):
    return pl.pallas_call(
        kfn,
        grid=(grid,),
        in_specs=in_specs,
        out_specs=out_spec,
        out_shape=jax.ShapeDtypeStruct(out_shape, out_dtype),
        scratch_shapes=list(scratch),
        compiler_params=pltpu.CompilerParams(
            dimension_semantics=("arbitrary",),
            vmem_limit_bytes=_VMEM),
    )(*args)


def kernel(x, fc1_w, fc1_b, t1_w, t1_b, t2_w, t2_b, t3_w, t3_b, t4_w, t4_b,
           bn1_g, bn1_b, bn2_g, bn2_b, bn3_g, bn3_b):
    B = x.shape[0]
    K = x.shape[1]

    full = lambda shape: pl.BlockSpec(shape, lambda j: tuple(0 for _ in shape))

    y1 = _pcall(
        _stage1_kernel, 8,
        [full((B, K)), full((K, 1024)), full((1, 1024)),
         pl.BlockSpec((1024, 1024), lambda j: (0, j))],
        [x, fc1_w, fc1_b.reshape(1, 1024), t1_w],
        (B, 8192),
        pl.BlockSpec((B, 1024), lambda j: (0, j)),
        scratch=[pltpu.VMEM((B, 1024), jnp.bfloat16)])

    bbA = B // 8
    bbC = B // 4
    w2f8 = (t2_w * 1024.0).astype(jnp.float8_e4m3fn)
    w3f8 = (t3_w * 1024.0).astype(jnp.float8_e4m3fn)
    wb3 = jnp.concatenate(
        [jnp.concatenate(
            [w3f8[:, (kh * 4 + kw) * 128:(kh * 4 + kw + 1) * 128]
             for _, kh in _P_SH[hp] for _, kw in _P_SH[wp]], axis=0)
         for hp, wp in ((0, 0), (0, 1), (1, 0), (1, 1))], axis=1)
    y4 = _pcall(
        functools.partial(_mid_kernel, B=B), 20,
        [full((B * 16, 512)), full((1, 512)), full((1, 512)),
         full((512, 4096)), full((1, 256)), full((1, 256)),
         full((1024, 512)), full((1, 128)), full((1, 128)),
         full((128, 128))],
        [y1.reshape(B * 16, 512), bn1_g.reshape(1, 512),
         bn1_b.reshape(1, 512), w2f8, bn2_g.reshape(1, 256),
         bn2_b.reshape(1, 256), wb3, bn3_g.reshape(1, 128),
         bn3_b.reshape(1, 128), t4_w.astype(jnp.bfloat16)],
        (B * 196, 16),
        pl.BlockSpec((bbC * 196, 16), lambda j: (jnp.maximum(j - 16, 0), 0)),
        out_dtype=jnp.bfloat16,
        scratch=[pltpu.VMEM((2, 512), jnp.float32),
                 pltpu.VMEM((bbA, 7, 7, 256), jnp.float32),
                 pltpu.VMEM((B, 49, 256), jnp.bfloat16),
                 pltpu.VMEM((2, 256), jnp.float32),
                 pltpu.VMEM((B, 196, 128), jnp.bfloat16),
                 pltpu.VMEM((2, 128), jnp.float32)])

    yl = y4.reshape(B, 196 * 16)
    img = _pcall(
        _stage5_kernel, 2,
        [pl.BlockSpec((B // 2, 196 * 16), lambda j: (j, 0)),
         full((196 * 16, 784)), full((1, 1))],
        [yl, jnp.asarray(_S4, dtype=jnp.bfloat16), t4_b.reshape(1, 1)],
        (B, 784),
        pl.BlockSpec((B // 2, 784), lambda j: (j, 0)))

    return img.reshape(B, 1, 28, 28)


# speedup vs baseline: 1.0623x; 1.0623x over previous
import functools

import jax
import jax.numpy as jnp
import numpy as np
from jax.experimental import pallas as pl
from jax.experimental.pallas import tpu as pltpu

_EPS = 1e-5
_VMEM = 60 * 1024 * 1024


def _make_scatter_matrix():
    s = np.zeros((196 * 16, 784), np.float32)
    for ih in range(14):
        for iw in range(14):
            for kh in range(4):
                for kw in range(4):
                    oh = 2 * ih + kh - 1
                    ow = 2 * iw + kw - 1
                    if 0 <= oh < 28 and 0 <= ow < 28:
                        s[(ih * 14 + iw) * 16 + kh * 4 + kw, oh * 28 + ow] = 1.0
    return s


_S4 = _make_scatter_matrix()


def _bn_scale_shift(x_full, gamma, beta):
    n = x_full.shape[0]
    mean = jnp.sum(x_full, axis=0, keepdims=True) * (1.0 / n)
    ex2 = jnp.sum(x_full * x_full, axis=0, keepdims=True) * (1.0 / n)
    inv = jax.lax.rsqrt(ex2 - mean * mean + _EPS)
    scale = inv * gamma
    return scale, beta - mean * scale


def _phase_taps(ph, hin):
    if ph == 0:
        return [(1, 0, 0, hin), (3, 1, 0, hin - 1)]
    return [(2, 0, 0, hin), (0, 0, 1, hin - 1)]


def _stage1_kernel(x_ref, w1_ref, b1_ref, t1_ref, o_ref, hb_ref):
    j = pl.program_id(0)

    @pl.when(j == 0)
    def _():
        h = jnp.dot(x_ref[...].astype(jnp.bfloat16),
                    w1_ref[...].astype(jnp.bfloat16),
                    preferred_element_type=jnp.float32) + b1_ref[...]
        hb_ref[...] = h.astype(jnp.bfloat16)

    o_ref[...] = jnp.dot(hb_ref[...], t1_ref[...].astype(jnp.bfloat16),
                         preferred_element_type=jnp.float32)


def _stage2_kernel(y1_ref, g_ref, b_ref, w2_ref, o_ref, wb_ref, ss_ref,
                   acc_ref, *, bb):
    i = pl.program_id(0)

    @pl.when(i == 0)
    def _():
        wb_ref[...] = (w2_ref[...] * 1024.0).astype(jnp.float8_e4m3fn)
        scale, shift = _bn_scale_shift(y1_ref[...], g_ref[...], b_ref[...])
        ss_ref[0:1, :] = scale
        ss_ref[1:2, :] = shift

    sl = y1_ref[pl.ds(i * (bb * 16), bb * 16), :]
    a = jnp.tanh(sl * ss_ref[0:1, :] + ss_ref[1:2, :])
    y = jnp.dot((a * 64.0).astype(jnp.float8_e4m3fn), wb_ref[...],
                preferred_element_type=jnp.float32) * (1.0 / (64.0 * 1024.0))
    acc_ref[...] = jnp.zeros_like(acc_ref)
    for kh in range(4):
        for kw in range(4):
            tap = y[:, (kh * 4 + kw) * 256:(kh * 4 + kw + 1) * 256]
            acc_ref[:, kh:kh + 4, kw:kw + 4, :] += tap.reshape(bb, 4, 4, 256)
    o_ref[...] = acc_ref[...].reshape(bb * 49, 256)


_P_SH = {0: ((0, 1), (1, 3)), 1: ((0, 2), (-1, 0))}


def _shift_rows(x, s, axis):
    if s == 0:
        return x
    n = x.shape[axis]
    pad = [(0, 0)] * x.ndim
    if s > 0:
        pad[axis] = (s, 0)
        sl = [slice(None)] * x.ndim
        sl[axis] = slice(0, n - s)
    else:
        pad[axis] = (0, -s)
        sl = [slice(None)] * x.ndim
        sl[axis] = slice(-s, n)
    return jnp.pad(x[tuple(sl)], pad)


def _stage3_kernel(y2_ref, g_ref, b_ref, w3_ref, o_ref, wb_ref, ss_ref, *, bb):
    i = pl.program_id(0)

    @pl.when(i == 0)
    def _():
        w3b = (w3_ref[...] * 1024.0).astype(jnp.float8_e4m3fn)
        for p, (hp, wp) in enumerate(((0, 0), (0, 1), (1, 0), (1, 1))):
            blocks = []
            for _, kh in _P_SH[hp]:
                for _, kw in _P_SH[wp]:
                    t = kh * 4 + kw
                    blocks.append(w3b[:, t * 128:(t + 1) * 128])
            wb_ref[:, p * 128:(p + 1) * 128] = jnp.concatenate(blocks, axis=0)
        scale, shift = _bn_scale_shift(y2_ref[...], g_ref[...], b_ref[...])
        ss_ref[0:1, :] = scale
        ss_ref[1:2, :] = shift

    sl = y2_ref[pl.ds(i * (bb * 49), bb * 49), :]
    a = (jnp.tanh(sl * ss_ref[0:1, :] + ss_ref[1:2, :])
         * 64.0).astype(jnp.float8_e4m3fn)
    x4 = a.reshape(bb, 7, 7, 256)
    q = []
    for hp in range(2):
        p = []
        for wp in range(2):
            acat = jnp.concatenate(
                [_shift_rows(_shift_rows(x4, sh, 1), sw, 2)
                 for sh, _ in _P_SH[hp] for sw, _ in _P_SH[wp]],
                axis=-1).reshape(bb * 49, 1024)
            pidx = hp * 2 + wp
            p.append((jnp.dot(acat, wb_ref[:, pidx * 128:(pidx + 1) * 128],
                              preferred_element_type=jnp.float32)
                      * (1.0 / (64.0 * 1024.0))).reshape(bb, 7, 7, 128))
        q.append(jnp.stack(p, axis=3).reshape(bb, 7, 14, 128))
    out = jnp.stack(q, axis=2).reshape(bb, 14, 14, 128)
    o_ref[...] = out.reshape(bb * 196, 128)


def _stage4_kernel(y3_ref, g_ref, b_ref, w4_ref, o_ref, wb_ref, ss_ref, *, bb):
    i = pl.program_id(0)

    @pl.when(i == 0)
    def _():
        wb_ref[...] = w4_ref[...].astype(jnp.bfloat16)
        scale, shift = _bn_scale_shift(y3_ref[...], g_ref[...], b_ref[...])
        ss_ref[0:1, :] = scale
        ss_ref[1:2, :] = shift

    sl = y3_ref[pl.ds(i * (bb * 196), bb * 196), :]
    a = jnp.tanh(sl * ss_ref[0:1, :] + ss_ref[1:2, :])
    y = jnp.dot(a.astype(jnp.bfloat16), wb_ref[...],
                preferred_element_type=jnp.float32)[:, :16]
    o_ref[...] = y.astype(jnp.bfloat16)


def _stage5_kernel(yl_ref, s4_ref, b4_ref, o_ref):
    z = jnp.dot(yl_ref[...], s4_ref[...],
                preferred_element_type=jnp.float32) + b4_ref[0, 0]
    ex = jnp.exp(-jnp.abs(z))
    o_ref[...] = jnp.where(z >= 0, 1.0 / (1.0 + ex), ex / (1.0 + ex))


def _pcall(kfn, grid, in_specs, args, out_shape, out_spec,
           out_dtype=jnp.float32, scratch=()):
    return pl.pallas_call(
        kfn,
        grid=(grid,),
        in_specs=in_specs,
        out_specs=out_spec,
        out_shape=jax.ShapeDtypeStruct(out_shape, out_dtype),
        scratch_shapes=list(scratch),
        compiler_params=pltpu.CompilerParams(
            dimension_semantics=("arbitrary",),
            vmem_limit_bytes=_VMEM),
    )(*args)


def kernel(x, fc1_w, fc1_b, t1_w, t1_b, t2_w, t2_b, t3_w, t3_b, t4_w, t4_b,
           bn1_g, bn1_b, bn2_g, bn2_b, bn3_g, bn3_b):
    B = x.shape[0]
    K = x.shape[1]

    full = lambda shape: pl.BlockSpec(shape, lambda j: tuple(0 for _ in shape))

    y1 = _pcall(
        _stage1_kernel, 8,
        [full((B, K)), full((K, 1024)), full((1, 1024)),
         pl.BlockSpec((1024, 1024), lambda j: (0, j))],
        [x, fc1_w, fc1_b.reshape(1, 1024), t1_w],
        (B, 8192),
        pl.BlockSpec((B, 1024), lambda j: (0, j)),
        scratch=[pltpu.VMEM((B, 1024), jnp.bfloat16)])

    bb2 = B // 4
    y2 = _pcall(
        functools.partial(_stage2_kernel, bb=bb2), 4,
        [full((B * 16, 512)), full((1, 512)), full((1, 512)),
         full((512, 4096))],
        [y1.reshape(B * 16, 512), bn1_g.reshape(1, 512),
         bn1_b.reshape(1, 512), t2_w],
        (B * 49, 256),
        pl.BlockSpec((bb2 * 49, 256), lambda j: (j, 0)),
        scratch=[pltpu.VMEM((512, 4096), jnp.float8_e4m3fn),
                 pltpu.VMEM((2, 512), jnp.float32),
                 pltpu.VMEM((bb2, 7, 7, 256), jnp.float32)])

    bb3 = B // 4
    y3 = _pcall(
        functools.partial(_stage3_kernel, bb=bb3), 4,
        [full((B * 49, 256)), full((1, 256)), full((1, 256)), full((256, 2048))],
        [y2, bn2_g.reshape(1, 256), bn2_b.reshape(1, 256), t3_w],
        (B * 196, 128),
        pl.BlockSpec((bb3 * 196, 128), lambda j: (j, 0)),
        scratch=[pltpu.VMEM((1024, 512), jnp.float8_e4m3fn),
                 pltpu.VMEM((2, 256), jnp.float32)])

    bb4 = B // 2
    y4 = _pcall(
        functools.partial(_stage4_kernel, bb=bb4), 2,
        [full((B * 196, 128)), full((1, 128)), full((1, 128)),
         full((128, 128))],
        [y3, bn3_g.reshape(1, 128), bn3_b.reshape(1, 128), t4_w],
        (B * 196, 16),
        pl.BlockSpec((bb4 * 196, 16), lambda j: (j, 0)),
        out_dtype=jnp.bfloat16,
        scratch=[pltpu.VMEM((128, 128), jnp.bfloat16),
                 pltpu.VMEM((2, 128), jnp.float32)])

    yl = y4.reshape(B, 196 * 16)
    img = _pcall(
        _stage5_kernel, 2,
        [pl.BlockSpec((B // 2, 196 * 16), lambda j: (j, 0)),
         full((196 * 16, 784)), full((1, 1))],
        [yl, jnp.asarray(_S4, dtype=jnp.bfloat16), t4_b.reshape(1, 1)],
        (B, 784),
        pl.BlockSpec((B // 2, 784), lambda j: (j, 0)))

    return img.reshape(B, 1, 28, 28)


# speedup vs baseline: 1.0638x; 1.0015x over previous
import functools

import jax
import jax.numpy as jnp
import numpy as np
from jax.experimental import pallas as pl
from jax.experimental.pallas import tpu as pltpu

_EPS = 1e-5
_VMEM = 60 * 1024 * 1024


def _make_scatter_matrix():
    s = np.zeros((196 * 16, 784), np.float32)
    for ih in range(14):
        for iw in range(14):
            for kh in range(4):
                for kw in range(4):
                    oh = 2 * ih + kh - 1
                    ow = 2 * iw + kw - 1
                    if 0 <= oh < 28 and 0 <= ow < 28:
                        s[(ih * 14 + iw) * 16 + kh * 4 + kw, oh * 28 + ow] = 1.0
    return s


_S4 = _make_scatter_matrix()


def _bn_scale_shift(x_full, gamma, beta):
    n = x_full.shape[0]
    mean = jnp.sum(x_full, axis=0, keepdims=True) * (1.0 / n)
    ex2 = jnp.sum(x_full * x_full, axis=0, keepdims=True) * (1.0 / n)
    inv = jax.lax.rsqrt(ex2 - mean * mean + _EPS)
    scale = inv * gamma
    return scale, beta - mean * scale


def _stage1_kernel(x_ref, w1_ref, b1_ref, t1_ref, o_ref, hb_ref):
    j = pl.program_id(0)

    @pl.when(j == 0)
    def _():
        h = jnp.dot(x_ref[...].astype(jnp.bfloat16),
                    w1_ref[...].astype(jnp.bfloat16),
                    preferred_element_type=jnp.float32) + b1_ref[...]
        hb_ref[...] = h.astype(jnp.bfloat16)

    o_ref[...] = jnp.dot(hb_ref[...], t1_ref[...].astype(jnp.bfloat16),
                         preferred_element_type=jnp.float32)


def _stage2_kernel(y1_ref, g_ref, b_ref, w2_ref, o_ref, wb_ref, ss_ref,
                   acc_ref, *, bb):
    i = pl.program_id(0)

    @pl.when(i == 0)
    def _():
        wb_ref[...] = (w2_ref[...] * 1024.0).astype(jnp.float8_e4m3fn)
        scale, shift = _bn_scale_shift(y1_ref[...], g_ref[...], b_ref[...])
        ss_ref[0:1, :] = scale
        ss_ref[1:2, :] = shift

    sl = y1_ref[pl.ds(i * (bb * 16), bb * 16), :]
    a = jnp.tanh(sl * ss_ref[0:1, :] + ss_ref[1:2, :])
    y = jnp.dot((a * 64.0).astype(jnp.float8_e4m3fn), wb_ref[...],
                preferred_element_type=jnp.float32) * (1.0 / (64.0 * 1024.0))
    acc_ref[...] = jnp.zeros_like(acc_ref)
    for kh in range(4):
        for kw in range(4):
            tap = y[:, (kh * 4 + kw) * 256:(kh * 4 + kw + 1) * 256]
            acc_ref[:, kh:kh + 4, kw:kw + 4, :] += tap.reshape(bb, 4, 4, 256)
    o_ref[...] = acc_ref[...].reshape(bb * 49, 256)


_P_SH = {0: ((0, 1), (1, 3)), 1: ((0, 2), (-1, 0))}


def _shift_rows(x, s, axis):
    if s == 0:
        return x
    n = x.shape[axis]
    pad = [(0, 0)] * x.ndim
    if s > 0:
        pad[axis] = (s, 0)
        sl = [slice(None)] * x.ndim
        sl[axis] = slice(0, n - s)
    else:
        pad[axis] = (0, -s)
        sl = [slice(None)] * x.ndim
        sl[axis] = slice(-s, n)
    return jnp.pad(x[tuple(sl)], pad)


def _stage3_kernel(y2_ref, g_ref, b_ref, w3_ref, o_ref, wb_ref, ss_ref, *, bb):
    i = pl.program_id(0)

    @pl.when(i == 0)
    def _():
        w3b = (w3_ref[...] * 1024.0).astype(jnp.float8_e4m3fn)
        for p, (hp, wp) in enumerate(((0, 0), (0, 1), (1, 0), (1, 1))):
            blocks = []
            for _, kh in _P_SH[hp]:
                for _, kw in _P_SH[wp]:
                    t = kh * 4 + kw
                    blocks.append(w3b[:, t * 128:(t + 1) * 128])
            wb_ref[:, p * 128:(p + 1) * 128] = jnp.concatenate(blocks, axis=0)
        scale, shift = _bn_scale_shift(y2_ref[...], g_ref[...], b_ref[...])
        ss_ref[0:1, :] = scale
        ss_ref[1:2, :] = shift

    sl = y2_ref[pl.ds(i * (bb * 49), bb * 49), :]
    a = (jnp.tanh(sl * ss_ref[0:1, :] + ss_ref[1:2, :])
         * 64.0).astype(jnp.float8_e4m3fn)
    x4 = a.reshape(bb, 7, 7, 256)
    q = []
    for hp in range(2):
        p = []
        for wp in range(2):
            acat = jnp.concatenate(
                [_shift_rows(_shift_rows(x4, sh, 1), sw, 2)
                 for sh, _ in _P_SH[hp] for sw, _ in _P_SH[wp]],
                axis=-1).reshape(bb * 49, 1024)
            pidx = hp * 2 + wp
            p.append((jnp.dot(acat, wb_ref[:, pidx * 128:(pidx + 1) * 128],
                              preferred_element_type=jnp.float32)
                      * (1.0 / (64.0 * 1024.0))).reshape(bb, 7, 7, 128))
        q.append(jnp.stack(p, axis=3).reshape(bb, 7, 14, 128))
    out = jnp.stack(q, axis=2).reshape(bb, 14, 14, 128)
    o_ref[...] = out.reshape(bb * 196, 128)


def _stage4_kernel(y3_ref, g_ref, b_ref, w4_ref, o_ref, wb_ref, ss_ref, *, bb):
    i = pl.program_id(0)

    @pl.when(i == 0)
    def _():
        wb_ref[...] = w4_ref[...].astype(jnp.bfloat16)
        scale, shift = _bn_scale_shift(y3_ref[...], g_ref[...], b_ref[...])
        ss_ref[0:1, :] = scale
        ss_ref[1:2, :] = shift

    sl = y3_ref[pl.ds(i * (bb * 196), bb * 196), :]
    a = jnp.tanh(sl * ss_ref[0:1, :] + ss_ref[1:2, :])
    y = jnp.dot(a.astype(jnp.bfloat16), wb_ref[...],
                preferred_element_type=jnp.float32)[:, :16]
    o_ref[...] = y.astype(jnp.bfloat16)


def _stage5_kernel(yl_ref, s4_ref, b4_ref, o_ref):
    z = jnp.dot(yl_ref[...], s4_ref[...],
                preferred_element_type=jnp.float32) + b4_ref[0, 0]
    ex = jnp.exp(-jnp.abs(z))
    o_ref[...] = jnp.where(z >= 0, 1.0 / (1.0 + ex), ex / (1.0 + ex))


def _pcall(kfn, grid, in_specs, args, out_shape, out_spec,
           out_dtype=jnp.float32, scratch=()):
    return pl.pallas_call(
        kfn,
        grid=(grid,),
        in_specs=in_specs,
        out_specs=out_spec,
        out_shape=jax.ShapeDtypeStruct(out_shape, out_dtype),
        scratch_shapes=list(scratch),
        compiler_params=pltpu.CompilerParams(
            dimension_semantics=("arbitrary",),
            vmem_limit_bytes=_VMEM),
    )(*args)


def kernel(x, fc1_w, fc1_b, t1_w, t1_b, t2_w, t2_b, t3_w, t3_b, t4_w, t4_b,
           bn1_g, bn1_b, bn2_g, bn2_b, bn3_g, bn3_b):
    B = x.shape[0]
    K = x.shape[1]

    full = lambda shape: pl.BlockSpec(shape, lambda j: tuple(0 for _ in shape))

    y1 = _pcall(
        _stage1_kernel, 8,
        [full((B, K)), full((K, 1024)), full((1, 1024)),
         pl.BlockSpec((1024, 1024), lambda j: (0, j))],
        [x, fc1_w, fc1_b.reshape(1, 1024), t1_w],
        (B, 8192),
        pl.BlockSpec((B, 1024), lambda j: (0, j)),
        scratch=[pltpu.VMEM((B, 1024), jnp.bfloat16)])

    bb2 = B // 4
    y2 = _pcall(
        functools.partial(_stage2_kernel, bb=bb2), 4,
        [full((B * 16, 512)), full((1, 512)), full((1, 512)),
         full((512, 4096))],
        [y1.reshape(B * 16, 512), bn1_g.reshape(1, 512),
         bn1_b.reshape(1, 512), t2_w],
        (B * 49, 256),
        pl.BlockSpec((bb2 * 49, 256), lambda j: (j, 0)),
        scratch=[pltpu.VMEM((512, 4096), jnp.float8_e4m3fn),
                 pltpu.VMEM((2, 512), jnp.float32),
                 pltpu.VMEM((bb2, 7, 7, 256), jnp.float32)])

    bb3 = B // 4
    y3 = _pcall(
        functools.partial(_stage3_kernel, bb=bb3), 4,
        [full((B * 49, 256)), full((1, 256)), full((1, 256)), full((256, 2048))],
        [y2, bn2_g.reshape(1, 256), bn2_b.reshape(1, 256), t3_w],
        (B * 196, 128),
        pl.BlockSpec((bb3 * 196, 128), lambda j: (j, 0)),
        scratch=[pltpu.VMEM((1024, 512), jnp.float8_e4m3fn),
                 pltpu.VMEM((2, 256), jnp.float32)])

    bb4 = B // 2
    y4 = _pcall(
        functools.partial(_stage4_kernel, bb=bb4), 2,
        [full((B * 196, 128)), full((1, 128)), full((1, 128)),
         full((128, 128))],
        [y3, bn3_g.reshape(1, 128), bn3_b.reshape(1, 128), t4_w],
        (B * 196, 16),
        pl.BlockSpec((bb4 * 196, 16), lambda j: (j, 0)),
        out_dtype=jnp.bfloat16,
        scratch=[pltpu.VMEM((128, 128), jnp.bfloat16),
                 pltpu.VMEM((2, 128), jnp.float32)])

    yl = y4.reshape(B, 196 * 16)
    img = _pcall(
        _stage5_kernel, 2,
        [pl.BlockSpec((B // 2, 196 * 16), lambda j: (j, 0)),
         full((196 * 16, 784)), full((1, 1))],
        [yl, jnp.asarray(_S4, dtype=jnp.bfloat16), t4_b.reshape(1, 1)],
        (B, 784),
        pl.BlockSpec((B // 2, 784), lambda j: (j, 0)))

    return img.reshape(B, 1, 28, 28)
